# Initial kernel scaffold; baseline (speedup 1.0000x reference)
import jax
import jax.numpy as jnp
from jax.experimental import pallas as pl


def kernel(R, Z, neighbor_idx, embeddings, W1, b1, W2, b2, W3, b3, scale, shift):
    raise NotImplementedError("write your pallas kernel here")


# speedup vs baseline: 43.9558x; 43.9558x over previous
import functools

import numpy as np
import jax
import jax.numpy as jnp
from jax import lax
from jax.experimental import pallas as pl
from jax.experimental.pallas import tpu as pltpu
from jax.experimental.pallas import tpu_sc as plsc

N_ATOMS = 10000
N_PAD = 10240
N_PAIRS = 320000
N_BASIS = 7
N_RADIAL = 5
N_SPECIES = 119
R_MIN = 0.5
R_MAX = 6.0
U1 = 512
U2 = 512

NCORES = 2
NSUB = 16
NW = NCORES * NSUB
GROUP = 128
NGROUPS = N_PAIRS // GROUP
EMB_COLS = 128
DCOL = 128
NPD = 8

BETTA = float(N_BASIS ** 2 / R_MAX ** 2)
RAD_NORM = float((2.0 * BETTA / np.pi) ** 0.25)
SHIFTS = [float(R_MIN + (R_MAX - R_MIN) / N_BASIS * b) for b in range(N_BASIS)]
INV_SQRT_B = float(1.0 / np.sqrt(N_BASIS))

_COMPS = ([()] + [(i,) for i in range(3)]
          + [(i, j) for i in range(3) for j in range(i, 3)]
          + [(i, j, k) for i in range(3) for j in range(i, 3) for k in range(j, 3)])
_COL_OF = {c: i for i, c in enumerate(_COMPS)}

_I2, _J2 = np.triu_indices(N_RADIAL)
_TRI3 = [(i, j, k) for i in range(N_RADIAL) for j in range(i, N_RADIAL)
         for k in range(j, N_RADIAL)]


def _mcol(r, idx):
    return r * 20 + _COL_OF[tuple(sorted(idx))]


def _acc(d, *cols):
    key = tuple(sorted(cols))
    d[key] = d.get(key, 0) + 1


def _emit(d, get):
    out = None
    for key, w in d.items():
        t = get(key[0])
        for c in key[1:]:
            t = t * get(c)
        if w != 1:
            t = t * float(w)
        out = t if out is None else out + t
    return out


def _gm_terms(get):
    gm = []
    for r in range(N_RADIAL):
        gm.append(get(_mcol(r, ())))
    for r, s in zip(_I2, _J2):
        d = {}
        for i in range(3):
            _acc(d, _mcol(r, (i,)), _mcol(s, (i,)))
        gm.append(_emit(d, get))
    for r, s in zip(_I2, _J2):
        d = {}
        for i in range(3):
            for j in range(3):
                _acc(d, _mcol(r, (i, j)), _mcol(s, (i, j)))
        gm.append(_emit(d, get))
    for r, s in zip(_I2, _J2):
        d = {}
        for i in range(3):
            for j in range(3):
                for k in range(3):
                    _acc(d, _mcol(r, (i, j, k)), _mcol(s, (i, j, k)))
        gm.append(_emit(d, get))
    for r, s, t in _TRI3:
        C = {}
        for j in range(3):
            for k in range(3):
                d = {}
                for i in range(3):
                    _acc(d, _mcol(r, (i, j)), _mcol(s, (i, k)))
                C[(j, k)] = _emit(d, get)
        out = None
        for j in range(3):
            for k in range(3):
                term = C[(j, k)] * get(_mcol(t, (j, k)))
                out = term if out is None else out + term
        gm.append(out)
    for r, s in zip(_I2, _J2):
        for t in range(N_RADIAL):
            d = {}
            for i in range(3):
                for j in range(3):
                    _acc(d, _mcol(r, (i,)), _mcol(s, (j,)), _mcol(t, (i, j)))
            gm.append(_emit(d, get))
    for r, s in zip(_I2, _J2):
        A = {}
        for k in range(3):
            for l in range(3):
                d = {}
                for i in range(3):
                    for j in range(3):
                        _acc(d, _mcol(r, (i, j, k)), _mcol(s, (i, j, l)))
                A[(k, l)] = _emit(d, get)
        for t in range(N_RADIAL):
            out = None
            for k in range(3):
                for l in range(3):
                    term = A[(k, l)] * get(_mcol(t, (k, l)))
                    out = term if out is None else out + term
            gm.append(out)
    for r in range(N_RADIAL):
        for s in range(N_RADIAL):
            B = {}
            for k in range(3):
                d = {}
                for i in range(3):
                    for j in range(3):
                        _acc(d, _mcol(r, (i, j, k)), _mcol(s, (i, j)))
                B[k] = _emit(d, get)
            for t in range(N_RADIAL):
                out = None
                for k in range(3):
                    term = B[k] * get(_mcol(t, (k,)))
                    out = term if out is None else out + term
                gm.append(out)
    assert len(gm) == 360
    return gm


def _lane_iota():
    return lax.broadcasted_iota(jnp.int32, (16,), 0)


def _rsqrt16(x):
    i = plsc.bitcast(x, jnp.int32)
    i = jnp.int32(0x5F3759DF) - lax.shift_right_logical(i, 1)
    y = plsc.bitcast(i, jnp.float32)
    for _ in range(3):
        y = y * (1.5 - 0.5 * x * y * y)
    return y


def _sin16(y):
    y2 = y * y
    return y * (0.9999966 + y2 * (-0.16664824 + y2 * (0.00830629 + y2 * -0.00018363)))


def _tile_range(total, wid):
    base, rem = total // NW, total % NW
    start = wid * base + jnp.minimum(wid, rem)
    count = base + jnp.where(wid < rem, 1, 0)
    return start, count


def _sc_geom_kernel(rx_h, ry_h, rz_h, z_h, ii_h, ij_h, emb_h, sca_h, shi_h,
                    pd_h, sg_h, hg_h,
                    rx_v, ry_v, rz_v, z_v, sca_v, shi_v,
                    iif_v, ijf_v, sidf_v, rows_v, pd_v, st_v, sem):
    cid = lax.axis_index("c")
    tid = lax.axis_index("s")
    wid = cid * NSUB + tid

    pltpu.sync_copy(rx_h, rx_v)
    pltpu.sync_copy(ry_h, ry_v)
    pltpu.sync_copy(rz_h, rz_v)
    pltpu.sync_copy(z_h, z_v)
    pltpu.sync_copy(sca_h, sca_v)
    pltpu.sync_copy(shi_h, shi_v)

    lane = _lane_iota()

    def _group(g, _):
        pb = g * GROUP
        pltpu.sync_copy(ii_h.at[pl.ds(pb, GROUP)], iif_v)
        pltpu.sync_copy(ij_h.at[pl.ds(pb, GROUP)], ijf_v)

        def _sid(sub, _):
            o = sub * 16
            vi = iif_v[pl.ds(o, 16)]
            vj = ijf_v[pl.ds(o, 16)]
            zi = plsc.load_gather(z_v, [vi])
            zj = plsc.load_gather(z_v, [vj])
            sidf_v[pl.ds(o, 16)] = zj * N_SPECIES + zi
            return 0

        lax.fori_loop(0, GROUP // 16, _sid, 0)

        pltpu.async_copy(emb_h.at[sidf_v], rows_v, sem).wait()

        def _pair16(sub, _):
            o = sub * 16
            fl = o + lane
            vi = iif_v[pl.ds(o, 16)]
            vj = ijf_v[pl.ds(o, 16)]
            xi = plsc.load_gather(rx_v, [vi])
            yi = plsc.load_gather(ry_v, [vi])
            zi_ = plsc.load_gather(rz_v, [vi])
            xj = plsc.load_gather(rx_v, [vj])
            yj = plsc.load_gather(ry_v, [vj])
            zj_ = plsc.load_gather(rz_v, [vj])
            dx = xi - xj
            dy = yi - yj
            dz = zi_ - zj_
            d2 = dx * dx + dy * dy + dz * dz + 1e-12
            rinv = _rsqrt16(d2)
            dr = d2 * rinv
            basis = []
            for b in range(N_BASIS):
                t = SHIFTS[b] - dr
                basis.append(RAD_NORM * jnp.exp(-BETTA * t * t))
            half = dr * float(np.pi / (2.0 * R_MAX))
            sn = _sin16(jnp.minimum(half, float(np.pi / 2.0)))
            cosv = 1.0 - 2.0 * sn * sn
            cut = jnp.where(dr < R_MAX, 0.5 * (cosv + 1.0), 0.0)
            cut = cut * INV_SQRT_B
            for r in range(N_RADIAL):
                acc = None
                for b in range(N_BASIS):
                    cc = jnp.full((16,), r * N_BASIS + b, jnp.int32)
                    cf = plsc.load_gather(rows_v, [fl, cc])
                    term = cf * basis[b]
                    acc = term if acc is None else acc + term
                pd_v[r, pl.ds(o, 16)] = acc * cut
            pd_v[5, pl.ds(o, 16)] = dx * rinv
            pd_v[6, pl.ds(o, 16)] = dy * rinv
            pd_v[7, pl.ds(o, 16)] = dz * rinv
            return 0

        lax.fori_loop(0, GROUP // 16, _pair16, 0)
        pltpu.sync_copy(pd_v, pd_h.at[:, pl.ds(pb, GROUP)])
        return 0

    g0, ng = _tile_range(NGROUPS, wid)
    lax.fori_loop(g0, g0 + ng, _group, 0)

    abase = wid * (N_PAD // NW)

    def _atoms(a, _):
        fl = a * 16 + lane
        za = plsc.load_gather(z_v, [abase + fl])
        plsc.store_scatter(st_v, [fl], plsc.load_gather(sca_v, [za]))
        plsc.store_scatter(st_v, [(N_PAD // NW) + fl], plsc.load_gather(shi_v, [za]))
        return 0

    lax.fori_loop(0, (N_PAD // NW) // 16, _atoms, 0)
    pltpu.sync_copy(st_v.at[pl.ds(0, N_PAD // NW)], sg_h.at[pl.ds(abase, N_PAD // NW)])
    pltpu.sync_copy(st_v.at[pl.ds(N_PAD // NW, N_PAD // NW)],
                    hg_h.at[pl.ds(abase, N_PAD // NW)])


def _sc_geom_call():
    return functools.partial(
        pl.kernel,
        out_type=(
            jax.ShapeDtypeStruct((NPD, N_PAIRS), jnp.float32),
            jax.ShapeDtypeStruct((N_PAD,), jnp.float32),
            jax.ShapeDtypeStruct((N_PAD,), jnp.float32),
        ),
        mesh=plsc.VectorSubcoreMesh(core_axis_name="c", subcore_axis_name="s",
                                    num_cores=NCORES, num_subcores=NSUB),
        scratch_types=[
            pltpu.VMEM((N_ATOMS,), jnp.float32),
            pltpu.VMEM((N_ATOMS,), jnp.float32),
            pltpu.VMEM((N_ATOMS,), jnp.float32),
            pltpu.VMEM((N_PAD,), jnp.int32),
            pltpu.VMEM((128,), jnp.float32),
            pltpu.VMEM((128,), jnp.float32),
            pltpu.VMEM((GROUP,), jnp.int32),
            pltpu.VMEM((GROUP,), jnp.int32),
            pltpu.VMEM((GROUP,), jnp.int32),
            pltpu.VMEM((GROUP, EMB_COLS), jnp.float32),
            pltpu.VMEM((NPD, GROUP), jnp.float32),
            pltpu.VMEM((2 * (N_PAD // NW),), jnp.float32),
            pltpu.SemaphoreType.DMA,
        ],
        compiler_params=pltpu.CompilerParams(needs_layout_passes=False),
    )


def _sc_moment_kernel(pd_h, ii_h, m_h,
                      pd_v, ii2_v, pay_v, m_sp):
    cid = lax.axis_index("c")
    tid = lax.axis_index("s")
    wid = cid * NSUB + tid

    zeros = jnp.zeros((16,), jnp.float32)

    def _zrow(r, _):
        for c in range(DCOL // 16):
            pay_v[r, pl.ds(c * 16, 16)] = zeros
        return 0

    lax.fori_loop(0, GROUP, _zrow, 0)

    zbase = tid * (N_PAD // NSUB)
    for q in range((N_PAD // NSUB) // GROUP):
        pltpu.sync_copy(pay_v, m_sp.at[pl.ds(zbase + q * GROUP, GROUP)])
    plsc.subcore_barrier()

    lane = _lane_iota()

    def _group(g, _):
        pb = g * GROUP
        pltpu.sync_copy(pd_h.at[:, pl.ds(pb, GROUP)], pd_v)
        pltpu.sync_copy(ii_h.at[pl.ds(pb, GROUP)], ii2_v.at[0])

        def _pair16(sub, _):
            o = sub * 16
            fl = o + lane
            rad = [pd_v[r, pl.ds(o, 16)] for r in range(N_RADIAL)]
            nx = pd_v[5, pl.ds(o, 16)]
            ny = pd_v[6, pl.ds(o, 16)]
            nz = pd_v[7, pl.ds(o, 16)]
            comp = [None, nx, ny, nz,
                    nx * nx, nx * ny, nx * nz, ny * ny, ny * nz, nz * nz]
            comp = comp + [comp[4] * nx, comp[4] * ny, comp[4] * nz,
                           comp[7] * nx, comp[8] * nx, comp[9] * nx,
                           comp[7] * ny, comp[7] * nz, comp[9] * ny, comp[9] * nz]
            for r in range(N_RADIAL):
                for m in range(20):
                    cc = jnp.full((16,), r * 20 + m, jnp.int32)
                    val = rad[r] if m == 0 else rad[r] * comp[m]
                    plsc.store_scatter(pay_v, [fl, cc], val)
            return 0

        lax.fori_loop(0, GROUP // 16, _pair16, 0)

        pltpu.sync_copy(pay_v, m_sp.at[ii2_v.at[0]], add=True)
        return 0

    g0, ng = _tile_range(NGROUPS, wid)
    lax.fori_loop(g0, g0 + ng, _group, 0)

    plsc.subcore_barrier()
    for q in range((N_PAD // NSUB) // GROUP):
        pltpu.sync_copy(m_sp.at[pl.ds(zbase + q * GROUP, GROUP)],
                        m_h.at[cid, pl.ds(zbase + q * GROUP, GROUP)])


def _sc_moment_call():
    return functools.partial(
        pl.kernel,
        out_type=jax.ShapeDtypeStruct((NCORES, N_PAD, DCOL), jnp.float32),
        mesh=plsc.VectorSubcoreMesh(core_axis_name="c", subcore_axis_name="s",
                                    num_cores=NCORES, num_subcores=NSUB),
        scratch_types=[
            pltpu.VMEM((NPD, GROUP), jnp.float32),
            pltpu.VMEM((1, 128), jnp.int32),
            pltpu.VMEM((GROUP, DCOL), jnp.float32),
            pltpu.VMEM_SHARED((N_PAD, DCOL), jnp.float32),
        ],
        compiler_params=pltpu.CompilerParams(needs_layout_passes=False),
    )


def _tc_dense_kernel(m_ref, sg_ref, hg_ref, w1_ref, b1_ref, w2_ref, b2_ref,
                     w3_ref, b3_ref, out_ref):
    msum = m_ref[0] + m_ref[1]
    cols = {}

    def get(c):
        if c not in cols:
            cols[c] = msum[:, c]
        return cols[c]

    gm = _gm_terms(get)
    gmat = jnp.stack(gm, axis=0)
    h = jnp.dot(w1_ref[...], gmat, preferred_element_type=jnp.float32) + b1_ref[...]
    h = h / (1.0 + jnp.exp(-h))
    h = jnp.dot(w2_ref[...], h, preferred_element_type=jnp.float32) + b2_ref[...]
    h = h / (1.0 + jnp.exp(-h))
    h = jnp.dot(w3_ref[...], h, preferred_element_type=jnp.float32) + b3_ref[...]
    out_ref[...] = sg_ref[...] * h + hg_ref[...]


_BLK = 128


def _tc_dense_call(m, sg, hg, w1t, b1, w2t, b2, w3t, b3):
    grid = (N_PAD // _BLK,)
    return pl.pallas_call(
        _tc_dense_kernel,
        grid=grid,
        in_specs=[
            pl.BlockSpec((NCORES, _BLK, DCOL), lambda i: (0, i, 0)),
            pl.BlockSpec((1, _BLK), lambda i: (0, i)),
            pl.BlockSpec((1, _BLK), lambda i: (0, i)),
            pl.BlockSpec((U1, 360), lambda i: (0, 0)),
            pl.BlockSpec((U1, 1), lambda i: (0, 0)),
            pl.BlockSpec((U2, U1), lambda i: (0, 0)),
            pl.BlockSpec((U2, 1), lambda i: (0, 0)),
            pl.BlockSpec((1, U2), lambda i: (0, 0)),
            pl.BlockSpec((1, 1), lambda i: (0, 0)),
        ],
        out_specs=pl.BlockSpec((1, _BLK), lambda i: (0, i)),
        out_shape=jax.ShapeDtypeStruct((1, N_PAD), jnp.float32),
    )(m, sg, hg, w1t, b1, w2t, b2, w3t, b3)


def kernel(R, Z, neighbor_idx, embeddings, W1, b1, W2, b2, W3, b3, scale, shift):
    rx = jnp.asarray(R[:, 0], jnp.float32)
    ry = jnp.asarray(R[:, 1], jnp.float32)
    rz = jnp.asarray(R[:, 2], jnp.float32)
    zp = jnp.pad(Z.astype(jnp.int32), (0, N_PAD - N_ATOMS))
    ii = neighbor_idx[0].astype(jnp.int32)
    ij = neighbor_idx[1].astype(jnp.int32)
    embf = embeddings.reshape(N_SPECIES * N_SPECIES, N_RADIAL * N_BASIS)
    embf = jnp.pad(embf, ((0, 0), (0, EMB_COLS - N_RADIAL * N_BASIS)))
    scaf = jnp.pad(scale[:, 0], (0, 128 - N_SPECIES))
    shif = jnp.pad(shift[:, 0], (0, 128 - N_SPECIES))

    pd, sg, hg = _sc_geom_call()(_sc_geom_kernel)(
        rx, ry, rz, zp, ii, ij, embf, scaf, shif)
    m = _sc_moment_call()(_sc_moment_kernel)(pd, ii)

    out = _tc_dense_call(
        m, sg.reshape(1, N_PAD), hg.reshape(1, N_PAD),
        W1.T, b1.reshape(U1, 1), W2.T, b2.reshape(U2, 1),
        W3.T, b3.reshape(1, 1))
    return out[0, :N_ATOMS, None]
